# Initial kernel scaffold; baseline (speedup 1.0000x reference)
import jax
import jax.numpy as jnp
from jax.experimental import pallas as pl


def kernel(src, emb_table, W1, b1, W2, b2):
    raise NotImplementedError("write your pallas kernel here")


# speedup vs baseline: 2.6371x; 2.6371x over previous
import functools

import jax
import jax.numpy as jnp
from jax import lax
from jax.experimental import pallas as pl
from jax.experimental.pallas import tpu as pltpu
from jax.experimental.pallas import tpu_sc as plsc

D = 64
B = 16384
HIST = 50
H = 128

NC, NS, L = 2, 16, 16
NW = NC * NS
BAGS_PER_W = B // NW
BAGS_PER_CHUNK = 2
ROWS_PER_CHUNK = BAGS_PER_CHUNK * HIST
CHUNKS = BAGS_PER_W // BAGS_PER_CHUNK
INV_HIST = 1.0 / HIST


def _embed_mean_body(src_hbm, table_hbm, out_hbm,
                     idx_v, buf0, buf1, out_v, sem0, sem1):
    wid = lax.axis_index("s") * NC + lax.axis_index("c")
    pltpu.sync_copy(src_hbm.at[wid], idx_v)

    bufs = (buf0, buf1)
    sems = (sem0, sem1)

    def gather_start(j, b):
        pltpu.async_copy(table_hbm.at[idx_v.at[j]], bufs[b], sems[b])

    def gather_wait(b):
        pltpu.make_async_copy(table_hbm.at[idx_v.at[0]], bufs[b], sems[b]).wait()

    def accumulate(j, b):
        buf = bufs[b]
        for bag in range(BAGS_PER_CHUNK):
            row0 = bag * HIST
            for q in range(D // L):
                col = pl.ds(q * L, L)
                acc_a = buf[row0, col]
                acc_b = buf[row0 + 1, col]
                for r in range(2, HIST, 2):
                    acc_a = acc_a + buf[row0 + r, col]
                    acc_b = acc_b + buf[row0 + r + 1, col]
                out_v[j * BAGS_PER_CHUNK + bag, col] = (acc_a + acc_b) * INV_HIST

    gather_start(0, 0)
    gather_start(1, 1)

    def loop_body(i, _):
        jj = i * 2
        for b in range(2):
            gather_wait(b)
            gather_start(jj + 2 + b, b)
            accumulate(jj + b, b)
        return ()

    lax.fori_loop(0, (CHUNKS - 2) // 2, loop_body, (), unroll=False)

    for b in range(2):
        gather_wait(b)
        accumulate(CHUNKS - 2 + b, b)

    pltpu.sync_copy(out_v, out_hbm.at[pl.ds(wid * BAGS_PER_W, BAGS_PER_W)])


def _make_embed_mean(interpret=False):
    mesh = plsc.VectorSubcoreMesh(
        core_axis_name="c", subcore_axis_name="s", num_cores=NC, num_subcores=NS
    )
    return pl.kernel(
        _embed_mean_body,
        out_type=jax.ShapeDtypeStruct((B, D), jnp.float32),
        mesh=mesh,
        scratch_types=[
            pltpu.VMEM((CHUNKS, ROWS_PER_CHUNK), jnp.int32),
            pltpu.VMEM((ROWS_PER_CHUNK, D), jnp.float32),
            pltpu.VMEM((ROWS_PER_CHUNK, D), jnp.float32),
            pltpu.VMEM((BAGS_PER_W, D), jnp.float32),
            pltpu.SemaphoreType.DMA,
            pltpu.SemaphoreType.DMA,
        ],
        compiler_params=pltpu.CompilerParams(use_tc_tiling_on_sc=False),
        interpret=interpret,
        name="embed_bag_mean_sc",
    )


def _mlp_body(x_ref, w1_ref, b1_ref, w2_ref, b2_ref, out_ref):
    x = x_ref[...]
    h = jnp.dot(x, w1_ref[...], preferred_element_type=jnp.float32) + b1_ref[...]
    h = jnp.maximum(h, 0.0)
    z = jnp.sum(h * w2_ref[...], axis=1, keepdims=True) + b2_ref[...]
    out_ref[...] = 1.0 / (1.0 + jnp.exp(-z))


MB = 2048


def _make_mlp(interpret=False):
    return pl.pallas_call(
        _mlp_body,
        grid=(B // MB,),
        in_specs=[
            pl.BlockSpec((MB, D), lambda i: (i, 0)),
            pl.BlockSpec((D, H), lambda i: (0, 0)),
            pl.BlockSpec((1, H), lambda i: (0, 0)),
            pl.BlockSpec((1, H), lambda i: (0, 0)),
            pl.BlockSpec((1, 1), lambda i: (0, 0)),
        ],
        out_specs=pl.BlockSpec((MB, 1), lambda i: (i, 0)),
        out_shape=jax.ShapeDtypeStruct((B, 1), jnp.float32),
        interpret=interpret,
        name="mlp_tc",
    )


@jax.jit
def _run(src, emb_table, W1, b1, W2, b2):
    src_r = jnp.reshape(src.astype(jnp.int32), (NW, CHUNKS, ROWS_PER_CHUNK))
    x_mean = _make_embed_mean()(src_r, emb_table)
    return _make_mlp()(
        x_mean, W1, b1.reshape(1, H), W2.reshape(1, H), b2.reshape(1, 1)
    )


def kernel(src, emb_table, W1, b1, W2, b2):
    return _run(src, emb_table, W1, b1, W2, b2)


# speedup vs baseline: 3.3520x; 1.2711x over previous
import functools

import jax
import jax.numpy as jnp
from jax import lax
from jax.experimental import pallas as pl
from jax.experimental.pallas import tpu as pltpu
from jax.experimental.pallas import tpu_sc as plsc

D = 64
B = 16384
HIST = 50
H = 128

NC, NS, L = 2, 16, 16
NW = NC * NS
BAGS_PER_W = B // NW
BAGS_PER_CHUNK = 2
ROWS_PER_CHUNK = BAGS_PER_CHUNK * HIST
CHUNKS = BAGS_PER_W // BAGS_PER_CHUNK
INV_HIST = 1.0 / HIST


def _embed_mean_body(src_hbm, table_hbm, out_hbm,
                     idx_v, buf0, buf1, out_v, sem0, sem1):
    wid = lax.axis_index("s") * NC + lax.axis_index("c")
    pltpu.sync_copy(src_hbm.at[wid], idx_v)

    bufs = (buf0, buf1)
    sems = (sem0, sem1)

    def gather_start(j, b):
        pltpu.async_copy(table_hbm.at[idx_v.at[j]], bufs[b], sems[b])

    def gather_wait(b):
        pltpu.make_async_copy(table_hbm.at[idx_v.at[0]], bufs[b], sems[b]).wait()

    def accumulate(j, b):
        buf = bufs[b]
        for bag in range(BAGS_PER_CHUNK):
            row0 = bag * HIST
            for q in range(D // L):
                col = pl.ds(q * L, L)
                acc_a = buf[row0, col]
                acc_b = buf[row0 + 1, col]
                for r in range(2, HIST, 2):
                    acc_a = acc_a + buf[row0 + r, col]
                    acc_b = acc_b + buf[row0 + r + 1, col]
                out_v[j * BAGS_PER_CHUNK + bag, col] = (acc_a + acc_b) * INV_HIST

    gather_start(0, 0)
    gather_start(1, 1)

    def loop_body(i, _):
        jj = i * 2
        for b in range(2):
            gather_wait(b)
            gather_start(jj + 2 + b, b)
            accumulate(jj + b, b)
        return ()

    lax.fori_loop(0, (CHUNKS - 2) // 2, loop_body, (), unroll=False)

    for b in range(2):
        gather_wait(b)
        accumulate(CHUNKS - 2 + b, b)

    pltpu.sync_copy(out_v, out_hbm.at[pl.ds(wid * BAGS_PER_W, BAGS_PER_W)])


def _make_embed_mean(interpret=False):
    mesh = plsc.VectorSubcoreMesh(
        core_axis_name="c", subcore_axis_name="s", num_cores=NC, num_subcores=NS
    )
    return pl.kernel(
        _embed_mean_body,
        out_type=jax.ShapeDtypeStruct((B, D), jnp.float32),
        mesh=mesh,
        scratch_types=[
            pltpu.VMEM((CHUNKS, ROWS_PER_CHUNK), jnp.int32),
            pltpu.VMEM((ROWS_PER_CHUNK, D), jnp.float32),
            pltpu.VMEM((ROWS_PER_CHUNK, D), jnp.float32),
            pltpu.VMEM((BAGS_PER_W, D), jnp.float32),
            pltpu.SemaphoreType.DMA,
            pltpu.SemaphoreType.DMA,
        ],
        compiler_params=pltpu.CompilerParams(use_tc_tiling_on_sc=False),
        interpret=interpret,
        name="embed_bag_mean_sc",
    )


V = 1000000
VB = 4096
TGRID = -(-V // VB)
VLAST = V - (TGRID - 1) * VB


def _transpose_body(tabT_ref, o_ref):
    y = tabT_ref[...].T
    z3 = jnp.reshape(y, (VB // 2, 2, D))
    o_ref[...] = jnp.concatenate([z3[:, 0, :], z3[:, 1, :]], axis=1)


def _make_transpose(interpret=False):
    return pl.pallas_call(
        _transpose_body,
        grid=(TGRID,),
        in_specs=[pl.BlockSpec((D, VB), lambda i: (0, i))],
        out_specs=pl.BlockSpec((VB // 2, 2 * D), lambda i: (i, 0)),
        out_shape=jax.ShapeDtypeStruct((V // 2, 2 * D), jnp.float32),
        interpret=interpret,
        name="table_detile_tc",
    )


def _mlp_body(x_ref, w1_ref, b1_ref, w2_ref, b2_ref, out_ref):
    x = x_ref[...]
    h = jnp.dot(x, w1_ref[...], preferred_element_type=jnp.float32) + b1_ref[...]
    h = jnp.maximum(h, 0.0)
    z = jnp.sum(h * w2_ref[...], axis=1, keepdims=True) + b2_ref[...]
    out_ref[...] = 1.0 / (1.0 + jnp.exp(-z))


MB = 2048


def _make_mlp(interpret=False):
    return pl.pallas_call(
        _mlp_body,
        grid=(B // MB,),
        in_specs=[
            pl.BlockSpec((MB, D), lambda i: (i, 0)),
            pl.BlockSpec((D, H), lambda i: (0, 0)),
            pl.BlockSpec((1, H), lambda i: (0, 0)),
            pl.BlockSpec((1, H), lambda i: (0, 0)),
            pl.BlockSpec((1, 1), lambda i: (0, 0)),
        ],
        out_specs=pl.BlockSpec((MB, 1), lambda i: (i, 0)),
        out_shape=jax.ShapeDtypeStruct((B, 1), jnp.float32),
        interpret=interpret,
        name="mlp_tc",
    )


@jax.jit
def _run(src, emb_table, W1, b1, W2, b2):
    src_r = jnp.reshape(src.astype(jnp.int32), (NW, CHUNKS, ROWS_PER_CHUNK))
    pairs = _make_transpose()(emb_table.T)
    tab_lin = jnp.reshape(pairs, (V, D))
    x_mean = _make_embed_mean()(src_r, tab_lin)
    return _make_mlp()(
        x_mean, W1, b1.reshape(1, H), W2.reshape(1, H), b2.reshape(1, 1)
    )


def kernel(src, emb_table, W1, b1, W2, b2):
    return _run(src, emb_table, W1, b1, W2, b2)


# speedup vs baseline: 3.9644x; 1.1827x over previous
import functools

import jax
import jax.numpy as jnp
from jax import lax
from jax.experimental import pallas as pl
from jax.experimental.pallas import tpu as pltpu
from jax.experimental.pallas import tpu_sc as plsc

D = 64
B = 16384
HIST = 50
H = 128

NC, NS, L = 2, 16, 16
NW = NC * NS
BAGS_PER_W = B // NW
BAGS_PER_CHUNK = 2
ROWS_PER_CHUNK = BAGS_PER_CHUNK * HIST
CHUNKS = BAGS_PER_W // BAGS_PER_CHUNK
INV_HIST = 1.0 / HIST


def _embed_mean_body(src_hbm, table_hbm, out_hbm,
                     idx_v, buf0, buf1, out_v, sem0, sem1):
    wid = lax.axis_index("s") * NC + lax.axis_index("c")
    pltpu.sync_copy(src_hbm.at[wid], idx_v)

    bufs = (buf0, buf1)
    sems = (sem0, sem1)

    def gather_start(j, b):
        pltpu.async_copy(table_hbm.at[idx_v.at[j]], bufs[b], sems[b])

    def gather_wait(b):
        pltpu.make_async_copy(table_hbm.at[idx_v.at[0]], bufs[b], sems[b]).wait()

    def accumulate(j, b):
        buf = bufs[b]
        for bag in range(BAGS_PER_CHUNK):
            row0 = bag * HIST
            for q in range(D // L):
                col = pl.ds(q * L, L)
                acc_a = buf[row0, col]
                acc_b = buf[row0 + 1, col]
                for r in range(2, HIST, 2):
                    acc_a = acc_a + buf[row0 + r, col]
                    acc_b = acc_b + buf[row0 + r + 1, col]
                out_v[j * BAGS_PER_CHUNK + bag, col] = (acc_a + acc_b) * INV_HIST

    gather_start(0, 0)
    gather_start(1, 1)

    def loop_body(i, _):
        jj = i * 2
        for b in range(2):
            gather_wait(b)
            gather_start(jj + 2 + b, b)
            accumulate(jj + b, b)
        return ()

    lax.fori_loop(0, (CHUNKS - 2) // 2, loop_body, (), unroll=False)

    for b in range(2):
        gather_wait(b)
        accumulate(CHUNKS - 2 + b, b)

    pltpu.sync_copy(out_v, out_hbm.at[pl.ds(wid * BAGS_PER_W, BAGS_PER_W)])


def _make_embed_mean(interpret=False):
    mesh = plsc.VectorSubcoreMesh(
        core_axis_name="c", subcore_axis_name="s", num_cores=NC, num_subcores=NS
    )
    return pl.kernel(
        _embed_mean_body,
        out_type=jax.ShapeDtypeStruct((B, D), jnp.float32),
        mesh=mesh,
        scratch_types=[
            pltpu.VMEM((CHUNKS, ROWS_PER_CHUNK), jnp.int32),
            pltpu.VMEM((ROWS_PER_CHUNK, D), jnp.float32),
            pltpu.VMEM((ROWS_PER_CHUNK, D), jnp.float32),
            pltpu.VMEM((BAGS_PER_W, D), jnp.float32),
            pltpu.SemaphoreType.DMA,
            pltpu.SemaphoreType.DMA,
        ],
        compiler_params=pltpu.CompilerParams(use_tc_tiling_on_sc=False),
        interpret=interpret,
        name="embed_bag_mean_sc",
    )


V = 1000000
VB = 4096
TGRID = -(-V // VB)
V_PAD = TGRID * VB


def _transpose_body(tabT_ref, o_ref):
    y = tabT_ref[...].T
    o_ref[...] = jnp.concatenate([y[: VB // 2, :], y[VB // 2 :, :]], axis=1)


def _make_transpose(interpret=False):
    return pl.pallas_call(
        _transpose_body,
        grid=(TGRID,),
        in_specs=[pl.BlockSpec((D, VB), lambda i: (0, i))],
        out_specs=pl.BlockSpec((VB // 2, 2 * D), lambda i: (i, 0)),
        out_shape=jax.ShapeDtypeStruct((V_PAD // 2, 2 * D), jnp.float32),
        interpret=interpret,
        name="table_detile_tc",
    )


def _mlp_body(x_ref, w1_ref, b1_ref, w2_ref, b2_ref, out_ref):
    x = x_ref[...]
    h = jnp.dot(x, w1_ref[...], preferred_element_type=jnp.float32) + b1_ref[...]
    h = jnp.maximum(h, 0.0)
    z = jnp.sum(h * w2_ref[...], axis=1, keepdims=True) + b2_ref[...]
    out_ref[...] = 1.0 / (1.0 + jnp.exp(-z))


MB = 2048


def _make_mlp(interpret=False):
    return pl.pallas_call(
        _mlp_body,
        grid=(B // MB,),
        in_specs=[
            pl.BlockSpec((MB, D), lambda i: (i, 0)),
            pl.BlockSpec((D, H), lambda i: (0, 0)),
            pl.BlockSpec((1, H), lambda i: (0, 0)),
            pl.BlockSpec((1, H), lambda i: (0, 0)),
            pl.BlockSpec((1, 1), lambda i: (0, 0)),
        ],
        out_specs=pl.BlockSpec((MB, 1), lambda i: (i, 0)),
        out_shape=jax.ShapeDtypeStruct((B, 1), jnp.float32),
        interpret=interpret,
        name="mlp_tc",
    )


@jax.jit
def _run(src, emb_table, W1, b1, W2, b2):
    src = src.astype(jnp.int32)
    src_l = (src & ~(VB - 1)) | ((src & (VB // 2 - 1)) << 1) | ((src >> 11) & 1)
    src_r = jnp.reshape(src_l, (NW, CHUNKS, ROWS_PER_CHUNK))
    pairs = _make_transpose()(emb_table.T)
    tab_lin = jnp.reshape(pairs, (V_PAD, D))
    x_mean = _make_embed_mean()(src_r, tab_lin)
    return _make_mlp()(
        x_mean, W1, b1.reshape(1, H), W2.reshape(1, H), b2.reshape(1, 1)
    )


def kernel(src, emb_table, W1, b1, W2, b2):
    return _run(src, emb_table, W1, b1, W2, b2)


# speedup vs baseline: 4.5608x; 1.1504x over previous
import functools

import jax
import jax.numpy as jnp
from jax import lax
from jax.experimental import pallas as pl
from jax.experimental.pallas import tpu as pltpu
from jax.experimental.pallas import tpu_sc as plsc

D = 64
B = 16384
HIST = 50
H = 128

NC, NS, L = 2, 16, 16
NW = NC * NS
BAGS_PER_W = B // NW
BAGS_PER_CHUNK = 2
ROWS_PER_CHUNK = BAGS_PER_CHUNK * HIST
CHUNKS = BAGS_PER_W // BAGS_PER_CHUNK
INV_HIST = 1.0 / HIST


def _embed_mean_body(src_hbm, table_hbm, out_hbm,
                     idx_v, buf0, buf1, out_v, sem0, sem1):
    wid = lax.axis_index("s") * NC + lax.axis_index("c")
    pltpu.sync_copy(src_hbm.at[wid], idx_v)

    bufs = (buf0, buf1)
    sems = (sem0, sem1)

    def gather_start(j, b):
        pltpu.async_copy(table_hbm.at[idx_v.at[j]], bufs[b], sems[b])

    def gather_wait(b):
        pltpu.make_async_copy(table_hbm.at[idx_v.at[0]], bufs[b], sems[b]).wait()

    def accumulate(j, b):
        buf = bufs[b]
        for bag in range(BAGS_PER_CHUNK):
            row0 = bag * HIST
            for q in range(D // L):
                col = pl.ds(q * L, L)
                acc_a = buf[row0, col]
                acc_b = buf[row0 + 1, col]
                for r in range(2, HIST, 2):
                    acc_a = acc_a + buf[row0 + r, col]
                    acc_b = acc_b + buf[row0 + r + 1, col]
                out_v[j * BAGS_PER_CHUNK + bag, col] = (acc_a + acc_b) * INV_HIST

    gather_start(0, 0)
    gather_start(1, 1)

    def loop_body(i, _):
        jj = i * 2
        for b in range(2):
            gather_wait(b)
            gather_start(jj + 2 + b, b)
            accumulate(jj + b, b)
        return ()

    lax.fori_loop(0, (CHUNKS - 2) // 2, loop_body, (), unroll=False)

    for b in range(2):
        gather_wait(b)
        accumulate(CHUNKS - 2 + b, b)

    pltpu.sync_copy(out_v, out_hbm.at[pl.ds(wid * BAGS_PER_W, BAGS_PER_W)])


def _make_embed_mean(interpret=False):
    mesh = plsc.VectorSubcoreMesh(
        core_axis_name="c", subcore_axis_name="s", num_cores=NC, num_subcores=NS
    )
    return pl.kernel(
        _embed_mean_body,
        out_type=jax.ShapeDtypeStruct((B, D), jnp.float32),
        mesh=mesh,
        scratch_types=[
            pltpu.VMEM((CHUNKS, ROWS_PER_CHUNK), jnp.int32),
            pltpu.VMEM((ROWS_PER_CHUNK, D), jnp.float32),
            pltpu.VMEM((ROWS_PER_CHUNK, D), jnp.float32),
            pltpu.VMEM((BAGS_PER_W, D), jnp.float32),
            pltpu.SemaphoreType.DMA,
            pltpu.SemaphoreType.DMA,
        ],
        compiler_params=pltpu.CompilerParams(use_tc_tiling_on_sc=False),
        interpret=interpret,
        name="embed_bag_mean_sc",
    )


V = 1000000
VB = 8192
TGRID = -(-V // VB)
V_PAD = TGRID * VB


def _transpose_body(tabT_ref, o_ref):
    y = tabT_ref[...].T
    o_ref[...] = jnp.concatenate([y[: VB // 2, :], y[VB // 2 :, :]], axis=1)


def _make_transpose(interpret=False):
    return pl.pallas_call(
        _transpose_body,
        grid=(TGRID,),
        in_specs=[pl.BlockSpec((D, VB), lambda i: (0, i))],
        out_specs=pl.BlockSpec((VB // 2, 2 * D), lambda i: (i, 0)),
        out_shape=jax.ShapeDtypeStruct((V_PAD // 2, 2 * D), jnp.float32),
        interpret=interpret,
        name="table_detile_tc",
    )


def _mlp_body(x_ref, w1_ref, b1_ref, w2_ref, b2_ref, out_ref):
    x = x_ref[...]
    h = jnp.dot(x, w1_ref[...], preferred_element_type=jnp.float32) + b1_ref[...]
    h = jnp.maximum(h, 0.0)
    z = jnp.sum(h * w2_ref[...], axis=1, keepdims=True) + b2_ref[...]
    out_ref[...] = 1.0 / (1.0 + jnp.exp(-z))


MB = 2048


def _make_mlp(interpret=False):
    return pl.pallas_call(
        _mlp_body,
        grid=(B // MB,),
        in_specs=[
            pl.BlockSpec((MB, D), lambda i: (i, 0)),
            pl.BlockSpec((D, H), lambda i: (0, 0)),
            pl.BlockSpec((1, H), lambda i: (0, 0)),
            pl.BlockSpec((1, H), lambda i: (0, 0)),
            pl.BlockSpec((1, 1), lambda i: (0, 0)),
        ],
        out_specs=pl.BlockSpec((MB, 1), lambda i: (i, 0)),
        out_shape=jax.ShapeDtypeStruct((B, 1), jnp.float32),
        interpret=interpret,
        name="mlp_tc",
    )


@jax.jit
def _run(src, emb_table, W1, b1, W2, b2):
    src = src.astype(jnp.int32)
    half_shift = (VB // 2).bit_length() - 1
    src_l = (
        (src & ~(VB - 1))
        | ((src & (VB // 2 - 1)) << 1)
        | ((src >> half_shift) & 1)
    )
    src_r = jnp.reshape(src_l, (NW, CHUNKS, ROWS_PER_CHUNK))
    pairs = _make_transpose()(emb_table.T)
    tab_lin = jnp.reshape(pairs, (V_PAD, D))
    x_mean = _make_embed_mean()(src_r, tab_lin)
    return _make_mlp()(
        x_mean, W1, b1.reshape(1, H), W2.reshape(1, H), b2.reshape(1, 1)
    )


def kernel(src, emb_table, W1, b1, W2, b2):
    return _run(src, emb_table, W1, b1, W2, b2)


# speedup vs baseline: 4.9268x; 1.0803x over previous
import functools

import jax
import jax.numpy as jnp
from jax import lax
from jax.experimental import pallas as pl
from jax.experimental.pallas import tpu as pltpu
from jax.experimental.pallas import tpu_sc as plsc

D = 64
B = 16384
HIST = 50
H = 128

NC, NS, L = 2, 16, 16
NW = NC * NS
BAGS_PER_W = B // NW
BAGS_PER_CHUNK = 2
ROWS_PER_CHUNK = BAGS_PER_CHUNK * HIST
CHUNKS = BAGS_PER_W // BAGS_PER_CHUNK
INV_HIST = 1.0 / HIST


def _embed_mean_body(src_hbm, table_hbm, out_hbm,
                     idx_v, buf0, buf1, out_v, sem0, sem1):
    wid = lax.axis_index("s") * NC + lax.axis_index("c")
    pltpu.sync_copy(src_hbm.at[wid], idx_v)

    bufs = (buf0, buf1)
    sems = (sem0, sem1)

    def gather_start(j, b):
        pltpu.async_copy(table_hbm.at[idx_v.at[j]], bufs[b], sems[b])

    def gather_wait(b):
        pltpu.make_async_copy(table_hbm.at[idx_v.at[0]], bufs[b], sems[b]).wait()

    def accumulate(j, b):
        buf = bufs[b]
        for bag in range(BAGS_PER_CHUNK):
            row0 = bag * HIST
            for q in range(D // L):
                col = pl.ds(q * L, L)
                acc_a = buf[row0, col]
                acc_b = buf[row0 + 1, col]
                for r in range(2, HIST, 2):
                    acc_a = acc_a + buf[row0 + r, col]
                    acc_b = acc_b + buf[row0 + r + 1, col]
                out_v[j * BAGS_PER_CHUNK + bag, col] = (acc_a + acc_b) * INV_HIST

    gather_start(0, 0)
    gather_start(1, 1)

    def loop_body(i, _):
        jj = i * 2
        for b in range(2):
            gather_wait(b)
            gather_start(jj + 2 + b, b)
            accumulate(jj + b, b)
        return ()

    lax.fori_loop(0, (CHUNKS - 2) // 2, loop_body, (), unroll=False)

    for b in range(2):
        gather_wait(b)
        accumulate(CHUNKS - 2 + b, b)

    pltpu.sync_copy(out_v, out_hbm.at[pl.ds(wid * BAGS_PER_W, BAGS_PER_W)])


def _make_embed_mean(interpret=False):
    mesh = plsc.VectorSubcoreMesh(
        core_axis_name="c", subcore_axis_name="s", num_cores=NC, num_subcores=NS
    )
    return pl.kernel(
        _embed_mean_body,
        out_type=jax.ShapeDtypeStruct((B, D), jnp.float32),
        mesh=mesh,
        scratch_types=[
            pltpu.VMEM((CHUNKS, ROWS_PER_CHUNK), jnp.int32),
            pltpu.VMEM((ROWS_PER_CHUNK, D), jnp.float32),
            pltpu.VMEM((ROWS_PER_CHUNK, D), jnp.float32),
            pltpu.VMEM((BAGS_PER_W, D), jnp.float32),
            pltpu.SemaphoreType.DMA,
            pltpu.SemaphoreType.DMA,
        ],
        compiler_params=pltpu.CompilerParams(use_tc_tiling_on_sc=False),
        interpret=interpret,
        name="embed_bag_mean_sc",
    )


V = 1000000
VB = 16384
TGRID = -(-V // VB)
V_PAD = TGRID * VB


def _transpose_body(tabT_ref, o_ref):
    y = tabT_ref[...].T
    o_ref[...] = jnp.concatenate([y[: VB // 2, :], y[VB // 2 :, :]], axis=1)


def _make_transpose(interpret=False):
    return pl.pallas_call(
        _transpose_body,
        grid=(TGRID,),
        in_specs=[pl.BlockSpec((D, VB), lambda i: (0, i))],
        out_specs=pl.BlockSpec((VB // 2, 2 * D), lambda i: (i, 0)),
        out_shape=jax.ShapeDtypeStruct((V_PAD // 2, 2 * D), jnp.float32),
        interpret=interpret,
        name="table_detile_tc",
    )


def _mlp_body(x_ref, w1_ref, b1_ref, w2_ref, b2_ref, out_ref):
    x = x_ref[...]
    h = jnp.dot(x, w1_ref[...], preferred_element_type=jnp.float32) + b1_ref[...]
    h = jnp.maximum(h, 0.0)
    z = jnp.sum(h * w2_ref[...], axis=1, keepdims=True) + b2_ref[...]
    out_ref[...] = 1.0 / (1.0 + jnp.exp(-z))


MB = 2048


def _make_mlp(interpret=False):
    return pl.pallas_call(
        _mlp_body,
        grid=(B // MB,),
        in_specs=[
            pl.BlockSpec((MB, D), lambda i: (i, 0)),
            pl.BlockSpec((D, H), lambda i: (0, 0)),
            pl.BlockSpec((1, H), lambda i: (0, 0)),
            pl.BlockSpec((1, H), lambda i: (0, 0)),
            pl.BlockSpec((1, 1), lambda i: (0, 0)),
        ],
        out_specs=pl.BlockSpec((MB, 1), lambda i: (i, 0)),
        out_shape=jax.ShapeDtypeStruct((B, 1), jnp.float32),
        interpret=interpret,
        name="mlp_tc",
    )


@jax.jit
def _run(src, emb_table, W1, b1, W2, b2):
    src = src.astype(jnp.int32)
    half_shift = (VB // 2).bit_length() - 1
    src_l = (
        (src & ~(VB - 1))
        | ((src & (VB // 2 - 1)) << 1)
        | ((src >> half_shift) & 1)
    )
    src_r = jnp.reshape(src_l, (NW, CHUNKS, ROWS_PER_CHUNK))
    pairs = _make_transpose()(emb_table.T)
    tab_lin = jnp.reshape(pairs, (V_PAD, D))
    x_mean = _make_embed_mean()(src_r, tab_lin)
    return _make_mlp()(
        x_mean, W1, b1.reshape(1, H), W2.reshape(1, H), b2.reshape(1, 1)
    )


def kernel(src, emb_table, W1, b1, W2, b2):
    return _run(src, emb_table, W1, b1, W2, b2)


# speedup vs baseline: 5.1037x; 1.0359x over previous
import functools

import jax
import jax.numpy as jnp
from jax import lax
from jax.experimental import pallas as pl
from jax.experimental.pallas import tpu as pltpu
from jax.experimental.pallas import tpu_sc as plsc

D = 64
B = 16384
HIST = 50
H = 128

NC, NS, L = 2, 16, 16
NW = NC * NS
BAGS_PER_W = B // NW
BAGS_PER_CHUNK = 2
ROWS_PER_CHUNK = BAGS_PER_CHUNK * HIST
CHUNKS = BAGS_PER_W // BAGS_PER_CHUNK
INV_HIST = 1.0 / HIST


def _embed_mean_body(src_hbm, table_hbm, out_hbm,
                     idx_v, buf0, buf1, out_v, sem0, sem1):
    wid = lax.axis_index("s") * NC + lax.axis_index("c")
    pltpu.sync_copy(src_hbm.at[wid], idx_v)

    bufs = (buf0, buf1)
    sems = (sem0, sem1)

    def gather_start(j, b):
        pltpu.async_copy(table_hbm.at[idx_v.at[j]], bufs[b], sems[b])

    def gather_wait(b):
        pltpu.make_async_copy(table_hbm.at[idx_v.at[0]], bufs[b], sems[b]).wait()

    def accumulate(j, b):
        buf = bufs[b]
        for bag in range(BAGS_PER_CHUNK):
            row0 = bag * HIST
            for q in range(D // L):
                col = pl.ds(q * L, L)
                acc_a = buf[row0, col]
                acc_b = buf[row0 + 1, col]
                for r in range(2, HIST, 2):
                    acc_a = acc_a + buf[row0 + r, col]
                    acc_b = acc_b + buf[row0 + r + 1, col]
                out_v[j * BAGS_PER_CHUNK + bag, col] = (acc_a + acc_b) * INV_HIST

    gather_start(0, 0)
    gather_start(1, 1)

    def loop_body(i, _):
        jj = i * 2
        for b in range(2):
            gather_wait(b)
            gather_start(jj + 2 + b, b)
            accumulate(jj + b, b)
        return ()

    lax.fori_loop(0, (CHUNKS - 2) // 2, loop_body, (), unroll=False)

    for b in range(2):
        gather_wait(b)
        accumulate(CHUNKS - 2 + b, b)

    pltpu.sync_copy(out_v, out_hbm.at[pl.ds(wid * BAGS_PER_W, BAGS_PER_W)])


def _make_embed_mean(interpret=False):
    mesh = plsc.VectorSubcoreMesh(
        core_axis_name="c", subcore_axis_name="s", num_cores=NC, num_subcores=NS
    )
    return pl.kernel(
        _embed_mean_body,
        out_type=jax.ShapeDtypeStruct((B, D), jnp.float32),
        mesh=mesh,
        scratch_types=[
            pltpu.VMEM((CHUNKS, ROWS_PER_CHUNK), jnp.int32),
            pltpu.VMEM((ROWS_PER_CHUNK, D), jnp.float32),
            pltpu.VMEM((ROWS_PER_CHUNK, D), jnp.float32),
            pltpu.VMEM((BAGS_PER_W, D), jnp.float32),
            pltpu.SemaphoreType.DMA,
            pltpu.SemaphoreType.DMA,
        ],
        compiler_params=pltpu.CompilerParams(use_tc_tiling_on_sc=False),
        interpret=interpret,
        name="embed_bag_mean_sc",
    )


V = 1000000
VB = 32768
TGRID = -(-V // VB)
V_PAD = TGRID * VB


def _transpose_body(tabT_ref, o_ref):
    y = tabT_ref[...].T
    o_ref[...] = jnp.concatenate([y[: VB // 2, :], y[VB // 2 :, :]], axis=1)


def _make_transpose(interpret=False):
    return pl.pallas_call(
        _transpose_body,
        grid=(TGRID,),
        in_specs=[pl.BlockSpec((D, VB), lambda i: (0, i))],
        out_specs=pl.BlockSpec((VB // 2, 2 * D), lambda i: (i, 0)),
        out_shape=jax.ShapeDtypeStruct((V_PAD // 2, 2 * D), jnp.float32),
        interpret=interpret,
        name="table_detile_tc",
    )


def _mlp_body(x_ref, w1_ref, b1_ref, w2_ref, b2_ref, out_ref):
    x = x_ref[...]
    h = jnp.dot(x, w1_ref[...], preferred_element_type=jnp.float32) + b1_ref[...]
    h = jnp.maximum(h, 0.0)
    z = jnp.sum(h * w2_ref[...], axis=1, keepdims=True) + b2_ref[...]
    out_ref[...] = 1.0 / (1.0 + jnp.exp(-z))


MB = 2048


def _make_mlp(interpret=False):
    return pl.pallas_call(
        _mlp_body,
        grid=(B // MB,),
        in_specs=[
            pl.BlockSpec((MB, D), lambda i: (i, 0)),
            pl.BlockSpec((D, H), lambda i: (0, 0)),
            pl.BlockSpec((1, H), lambda i: (0, 0)),
            pl.BlockSpec((1, H), lambda i: (0, 0)),
            pl.BlockSpec((1, 1), lambda i: (0, 0)),
        ],
        out_specs=pl.BlockSpec((MB, 1), lambda i: (i, 0)),
        out_shape=jax.ShapeDtypeStruct((B, 1), jnp.float32),
        interpret=interpret,
        name="mlp_tc",
    )


@jax.jit
def _run(src, emb_table, W1, b1, W2, b2):
    src = src.astype(jnp.int32)
    half_shift = (VB // 2).bit_length() - 1
    src_l = (
        (src & ~(VB - 1))
        | ((src & (VB // 2 - 1)) << 1)
        | ((src >> half_shift) & 1)
    )
    src_r = jnp.reshape(src_l, (NW, CHUNKS, ROWS_PER_CHUNK))
    pairs = _make_transpose()(emb_table.T)
    tab_lin = jnp.reshape(pairs, (V_PAD, D))
    x_mean = _make_embed_mean()(src_r, tab_lin)
    return _make_mlp()(
        x_mean, W1, b1.reshape(1, H), W2.reshape(1, H), b2.reshape(1, 1)
    )


def kernel(src, emb_table, W1, b1, W2, b2):
    return _run(src, emb_table, W1, b1, W2, b2)


# speedup vs baseline: 5.2407x; 1.0268x over previous
import functools

import jax
import jax.numpy as jnp
from jax import lax
from jax.experimental import pallas as pl
from jax.experimental.pallas import tpu as pltpu
from jax.experimental.pallas import tpu_sc as plsc

D = 64
B = 16384
HIST = 50
H = 128

NC, NS, L = 2, 16, 16
NW = NC * NS
BAGS_PER_W = B // NW
BAGS_PER_CHUNK = 2
ROWS_PER_CHUNK = BAGS_PER_CHUNK * HIST
CHUNKS = BAGS_PER_W // BAGS_PER_CHUNK
INV_HIST = 1.0 / HIST


def _embed_mean_body(src_hbm, table_hbm, out_hbm,
                     idx_v, buf0, buf1, out_v, sem0, sem1):
    wid = lax.axis_index("s") * NC + lax.axis_index("c")
    pltpu.sync_copy(src_hbm.at[wid], idx_v)

    bufs = (buf0, buf1)
    sems = (sem0, sem1)

    def gather_start(j, b):
        pltpu.async_copy(table_hbm.at[idx_v.at[j]], bufs[b], sems[b])

    def gather_wait(b):
        pltpu.make_async_copy(table_hbm.at[idx_v.at[0]], bufs[b], sems[b]).wait()

    def accumulate(j, b):
        buf = bufs[b]
        for bag in range(BAGS_PER_CHUNK):
            row0 = bag * HIST
            for q in range(D // L):
                col = pl.ds(q * L, L)
                acc_a = buf[row0, col]
                acc_b = buf[row0 + 1, col]
                for r in range(2, HIST, 2):
                    acc_a = acc_a + buf[row0 + r, col]
                    acc_b = acc_b + buf[row0 + r + 1, col]
                out_v[j * BAGS_PER_CHUNK + bag, col] = (acc_a + acc_b) * INV_HIST

    gather_start(0, 0)
    gather_start(1, 1)

    def loop_body(i, _):
        jj = i * 2
        for b in range(2):
            gather_wait(b)
            gather_start(jj + 2 + b, b)
            accumulate(jj + b, b)
        return ()

    lax.fori_loop(0, (CHUNKS - 2) // 2, loop_body, (), unroll=False)

    for b in range(2):
        gather_wait(b)
        accumulate(CHUNKS - 2 + b, b)

    pltpu.sync_copy(out_v, out_hbm.at[pl.ds(wid * BAGS_PER_W, BAGS_PER_W)])


def _make_embed_mean(interpret=False):
    mesh = plsc.VectorSubcoreMesh(
        core_axis_name="c", subcore_axis_name="s", num_cores=NC, num_subcores=NS
    )
    return pl.kernel(
        _embed_mean_body,
        out_type=jax.ShapeDtypeStruct((B, D), jnp.float32),
        mesh=mesh,
        scratch_types=[
            pltpu.VMEM((CHUNKS, ROWS_PER_CHUNK), jnp.int32),
            pltpu.VMEM((ROWS_PER_CHUNK, D), jnp.float32),
            pltpu.VMEM((ROWS_PER_CHUNK, D), jnp.float32),
            pltpu.VMEM((BAGS_PER_W, D), jnp.float32),
            pltpu.SemaphoreType.DMA,
            pltpu.SemaphoreType.DMA,
        ],
        compiler_params=pltpu.CompilerParams(use_tc_tiling_on_sc=False),
        interpret=interpret,
        name="embed_bag_mean_sc",
    )


V = 1000000
VB = 32768
TGRID = -(-V // VB)
V_PAD = TGRID * VB


def _transpose_body(tabT_ref, o_ref):
    y = tabT_ref[...].T
    o_ref[...] = jnp.concatenate([y[: VB // 2, :], y[VB // 2 :, :]], axis=1)


def _make_transpose(interpret=False):
    return pl.pallas_call(
        _transpose_body,
        grid=(TGRID,),
        in_specs=[pl.BlockSpec((D, VB), lambda i: (0, i))],
        out_specs=pl.BlockSpec((VB // 2, 2 * D), lambda i: (i, 0)),
        out_shape=jax.ShapeDtypeStruct((V_PAD // 2, 2 * D), jnp.float32),
        interpret=interpret,
        name="table_detile_tc",
    )


def _mlp_body(x2_ref, w1_ref, b1_ref, w2_ref, b2_ref, out_ref):
    x2 = x2_ref[...]
    w1 = w1_ref[...]
    b1 = b1_ref[...]
    w2 = w2_ref[...]
    b2 = b2_ref[...]

    def head(x):
        h = jnp.maximum(jnp.dot(x, w1, preferred_element_type=jnp.float32) + b1, 0.0)
        z = jnp.sum(h * w2, axis=1, keepdims=True) + b2
        return 1.0 / (1.0 + jnp.exp(-z))

    out_ref[...] = jnp.concatenate(
        [head(x2[:, :D]), head(x2[:, D:])], axis=1
    )


MB2 = 2048


def _make_mlp(interpret=False):
    return pl.pallas_call(
        _mlp_body,
        grid=(B // 2 // MB2,),
        in_specs=[
            pl.BlockSpec((MB2, 2 * D), lambda i: (i, 0)),
            pl.BlockSpec((D, H), lambda i: (0, 0)),
            pl.BlockSpec((1, H), lambda i: (0, 0)),
            pl.BlockSpec((1, H), lambda i: (0, 0)),
            pl.BlockSpec((1, 1), lambda i: (0, 0)),
        ],
        out_specs=pl.BlockSpec((MB2, 2), lambda i: (i, 0)),
        out_shape=jax.ShapeDtypeStruct((B // 2, 2), jnp.float32),
        interpret=interpret,
        name="mlp_tc",
    )


@jax.jit
def _run(src, emb_table, W1, b1, W2, b2):
    src = src.astype(jnp.int32)
    half_shift = (VB // 2).bit_length() - 1
    src_l = (
        (src & ~(VB - 1))
        | ((src & (VB // 2 - 1)) << 1)
        | ((src >> half_shift) & 1)
    )
    src_r = jnp.reshape(src_l, (NW, CHUNKS, ROWS_PER_CHUNK))
    pairs = _make_transpose()(emb_table.T)
    tab_lin = jnp.reshape(pairs, (V_PAD, D))
    x_mean = _make_embed_mean()(src_r, tab_lin)
    x2 = jnp.reshape(x_mean, (B // 2, 2 * D))
    out2 = _make_mlp()(
        x2, W1, b1.reshape(1, H), W2.reshape(1, H), b2.reshape(1, 1)
    )
    return jnp.reshape(out2, (B, 1))


def kernel(src, emb_table, W1, b1, W2, b2):
    return _run(src, emb_table, W1, b1, W2, b2)
